# baseline (device time: 28083 ns/iter reference)
import jax
import jax.numpy as jnp
from jax import lax
from jax.experimental import pallas as pl
from jax.experimental.pallas import tpu as pltpu

N_DEV = 16
N_ROUNDS = 4
B, SQ, D_MODEL = 2, 128, 512
HQ, DH = 4, 64
SKV_LOC = 128
BLK = 64
COLS = HQ * DH
PROWS = SQ + 8


def kernel(x, Wq, K_ext, V_ext, Wo):
    def body(x_ref, wq_ref, k_ref, v_ref, wo_ref, out_ref,
             acc, comm, send_sems, recv_sems):
        my = lax.axis_index("i")
        partners = [my ^ (1 << k) for k in range(N_ROUNDS)]

        row_blk = lax.broadcasted_iota(jnp.int32, (SQ, SKV_LOC), 0) // BLK
        col_blk = lax.broadcasted_iota(jnp.int32, (SQ, SKV_LOC), 1) // BLK + 2 * my
        mask = (row_blk == col_blk) | (col_blk == 0) | ((row_blk + col_blk) % 3 == 0)
        ones_row = jnp.ones((1, SKV_LOC), jnp.bfloat16)

        def compute_partials(b):
            acc[b, pl.ds(SQ, 8), :] = jnp.zeros((8, COLS), jnp.float32)
            q_b = jnp.dot(x_ref[b].astype(jnp.bfloat16),
                          wq_ref[...].astype(jnp.bfloat16),
                          preferred_element_type=jnp.float32)
            for h in range(HQ):
                q_bh = q_b[:, h * DH:(h + 1) * DH].astype(jnp.bfloat16)
                k_bh = k_ref[b, :, h, :].astype(jnp.bfloat16)
                v_bh = v_ref[b, :, h, :].astype(jnp.bfloat16)
                scores = lax.dot_general(
                    q_bh, k_bh, (((1,), (1,)), ((), ())),
                    preferred_element_type=jnp.float32) * 0.125
                w = jnp.where(mask, jnp.exp(scores), 0.0).astype(jnp.bfloat16)
                acc[b, pl.ds(0, SQ), pl.ds(h * DH, DH)] = jnp.dot(
                    w, v_bh, preferred_element_type=jnp.float32)
                acc[b, pl.ds(SQ + h, 1), pl.ds(0, SQ)] = lax.dot_general(
                    ones_row, w, (((1,), (1,)), ((), ())),
                    preferred_element_type=jnp.float32)

        def exchange(k, half):
            r = pltpu.make_async_remote_copy(
                src_ref=acc.at[half], dst_ref=comm.at[k, half],
                send_sem=send_sems.at[k, half], recv_sem=recv_sems.at[k, half],
                device_id=(partners[k],), device_id_type=pl.DeviceIdType.MESH)
            r.start()
            return r

        def finalize(b):
            s_cols = jnp.transpose(
                acc[b, pl.ds(SQ, 8), pl.ds(0, SQ)])
            out_b = jnp.zeros((SQ, D_MODEL), jnp.float32)
            for h in range(HQ):
                ctx = acc[b, pl.ds(0, SQ), pl.ds(h * DH, DH)] / s_cols[:, h:h + 1]
                out_b = out_b + jnp.dot(ctx.astype(jnp.bfloat16),
                                        wo_ref[pl.ds(h * DH, DH), :].astype(jnp.bfloat16),
                                        preferred_element_type=jnp.float32)
            out_ref[b] = out_b

        compute_partials(0)

        barrier = pltpu.get_barrier_semaphore()
        for p in partners:
            pl.semaphore_signal(barrier, inc=1, device_id=(p,),
                                device_id_type=pl.DeviceIdType.MESH)
        pl.semaphore_wait(barrier, N_ROUNDS)

        rdma_a = exchange(0, 0)
        compute_partials(1)
        rdma_b = exchange(0, 1)

        for k in range(N_ROUNDS):
            rdma_a.wait()
            acc[0] += comm[k, 0]
            if k + 1 < N_ROUNDS:
                rdma_a = exchange(k + 1, 0)
            rdma_b.wait()
            acc[1] += comm[k, 1]
            if k + 1 < N_ROUNDS:
                rdma_b = exchange(k + 1, 1)

        finalize(0)
        finalize(1)

    return pl.pallas_call(
        body,
        out_shape=jax.ShapeDtypeStruct((B, SQ, D_MODEL), jnp.float32),
        in_specs=[pl.BlockSpec(memory_space=pltpu.VMEM)] * 5,
        out_specs=pl.BlockSpec(memory_space=pltpu.VMEM),
        scratch_shapes=[
            pltpu.VMEM((B, PROWS, COLS), jnp.float32),
            pltpu.VMEM((N_ROUNDS, B, PROWS, COLS), jnp.float32),
            pltpu.SemaphoreType.DMA((N_ROUNDS, B)),
            pltpu.SemaphoreType.DMA((N_ROUNDS, B)),
        ],
        compiler_params=pltpu.CompilerParams(collective_id=0),
    )(x, Wq, K_ext, V_ext, Wo)


# device time: 22548 ns/iter; 1.2455x vs baseline; 1.2455x over previous
import jax
import jax.numpy as jnp
from jax import lax
from jax.experimental import pallas as pl
from jax.experimental.pallas import tpu as pltpu

N_DEV = 16
N_ROUNDS = 4
B, SQ, D_MODEL = 2, 128, 512
HQ, DH = 4, 64
SKV_LOC = 128
BLK = 64
COLS = HQ * DH
PROWS = SQ + 8


def kernel(x, Wq, K_ext, V_ext, Wo):
    def body(x_ref, wq_ref, k_ref, v_ref, wo_ref, out_ref,
             acc, stage, comm, send_sems, recv_sems):
        my = lax.axis_index("i")
        partners = [my ^ (1 << k) for k in range(N_ROUNDS)]

        row_blk = lax.broadcasted_iota(jnp.int32, (SQ, SKV_LOC), 0) // BLK
        col_blk = lax.broadcasted_iota(jnp.int32, (SQ, SKV_LOC), 1) // BLK + 2 * my
        mask = (row_blk == col_blk) | (col_blk == 0) | ((row_blk + col_blk) % 3 == 0)
        ones_row = jnp.ones((1, SKV_LOC), jnp.bfloat16)

        def compute_partials(b):
            acc[b, pl.ds(SQ, 8), :] = jnp.zeros((8, COLS), jnp.float32)
            q_b = jnp.dot(x_ref[b].astype(jnp.bfloat16),
                          wq_ref[...].astype(jnp.bfloat16),
                          preferred_element_type=jnp.float32)
            for h in range(HQ):
                q_bh = q_b[:, h * DH:(h + 1) * DH].astype(jnp.bfloat16)
                k_bh = k_ref[b, :, h, :].astype(jnp.bfloat16)
                v_bh = v_ref[b, :, h, :].astype(jnp.bfloat16)
                scores = lax.dot_general(
                    q_bh, k_bh, (((1,), (1,)), ((), ())),
                    preferred_element_type=jnp.float32) * 0.125
                w = jnp.where(mask, jnp.exp(scores), 0.0).astype(jnp.bfloat16)
                acc[b, pl.ds(0, SQ), pl.ds(h * DH, DH)] = jnp.dot(
                    w, v_bh, preferred_element_type=jnp.float32)
                acc[b, pl.ds(SQ + h, 1), pl.ds(0, SQ)] = lax.dot_general(
                    ones_row, w, (((1,), (1,)), ((), ())),
                    preferred_element_type=jnp.float32)

        def exchange(k, half):
            stage[half] = acc[half].astype(jnp.bfloat16)
            r = pltpu.make_async_remote_copy(
                src_ref=stage.at[half], dst_ref=comm.at[k, half],
                send_sem=send_sems.at[k, half], recv_sem=recv_sems.at[k, half],
                device_id=(partners[k],), device_id_type=pl.DeviceIdType.MESH)
            r.start()
            return r

        def finalize(b):
            s_cols = jnp.transpose(
                acc[b, pl.ds(SQ, 8), pl.ds(0, SQ)])
            out_b = jnp.zeros((SQ, D_MODEL), jnp.float32)
            for h in range(HQ):
                ctx = acc[b, pl.ds(0, SQ), pl.ds(h * DH, DH)] / s_cols[:, h:h + 1]
                out_b = out_b + jnp.dot(ctx.astype(jnp.bfloat16),
                                        wo_ref[pl.ds(h * DH, DH), :].astype(jnp.bfloat16),
                                        preferred_element_type=jnp.float32)
            out_ref[b] = out_b

        compute_partials(0)

        barrier = pltpu.get_barrier_semaphore()
        for p in partners:
            pl.semaphore_signal(barrier, inc=1, device_id=(p,),
                                device_id_type=pl.DeviceIdType.MESH)
        pl.semaphore_wait(barrier, N_ROUNDS)

        rdma_a = exchange(0, 0)
        compute_partials(1)
        rdma_b = exchange(0, 1)

        for k in range(N_ROUNDS):
            rdma_a.wait()
            acc[0] += comm[k, 0].astype(jnp.float32)
            if k + 1 < N_ROUNDS:
                rdma_a = exchange(k + 1, 0)
            else:
                finalize(0)
            rdma_b.wait()
            acc[1] += comm[k, 1].astype(jnp.float32)
            if k + 1 < N_ROUNDS:
                rdma_b = exchange(k + 1, 1)
            else:
                finalize(1)

    return pl.pallas_call(
        body,
        out_shape=jax.ShapeDtypeStruct((B, SQ, D_MODEL), jnp.float32),
        in_specs=[pl.BlockSpec(memory_space=pltpu.VMEM)] * 5,
        out_specs=pl.BlockSpec(memory_space=pltpu.VMEM),
        scratch_shapes=[
            pltpu.VMEM((B, PROWS, COLS), jnp.float32),
            pltpu.VMEM((B, PROWS, COLS), jnp.bfloat16),
            pltpu.VMEM((N_ROUNDS, B, PROWS, COLS), jnp.bfloat16),
            pltpu.SemaphoreType.DMA((N_ROUNDS, B)),
            pltpu.SemaphoreType.DMA((N_ROUNDS, B)),
        ],
        compiler_params=pltpu.CompilerParams(collective_id=0),
    )(x, Wq, K_ext, V_ext, Wo)


# device time: 20750 ns/iter; 1.3534x vs baseline; 1.0867x over previous
import jax
import jax.numpy as jnp
from jax import lax
from jax.experimental import pallas as pl
from jax.experimental.pallas import tpu as pltpu

N_DEV = 16
N_ROUNDS = 4
B, SQ, D_MODEL = 2, 128, 512
HQ, DH = 4, 64
SKV_LOC = 128
BLK = 64
COLS = HQ * DH
PROWS = SQ + 8
QW = 128
NQ = 4


def kernel(x, Wq, K_ext, V_ext, Wo):
    def body(x_ref, wq_ref, k_ref, v_ref, wo_ref, out_ref,
             acc, stage, comm, send_sems, recv_sems):
        my = lax.axis_index("i")
        partners = [my ^ (1 << k) for k in range(N_ROUNDS)]

        row_blk = lax.broadcasted_iota(jnp.int32, (SQ, SKV_LOC), 0) // BLK
        col_blk = lax.broadcasted_iota(jnp.int32, (SQ, SKV_LOC), 1) // BLK + 2 * my
        mask = (row_blk == col_blk) | (col_blk == 0) | ((row_blk + col_blk) % 3 == 0)
        ones_row = jnp.ones((1, SKV_LOC), jnp.bfloat16)
        expand = (lax.broadcasted_iota(jnp.int32, (2, QW), 1) // DH
                  == lax.broadcasted_iota(jnp.int32, (2, QW), 0)
                  ).astype(jnp.bfloat16)

        def compute_head(b, h, q_b):
            q_bh = q_b[:, h * DH:(h + 1) * DH].astype(jnp.bfloat16)
            k_bh = k_ref[b, :, h, :].astype(jnp.bfloat16)
            v_bh = v_ref[b, :, h, :].astype(jnp.bfloat16)
            scores = lax.dot_general(
                q_bh, k_bh, (((1,), (1,)), ((), ())),
                preferred_element_type=jnp.float32) * 0.125
            w = jnp.where(mask, jnp.exp(scores), 0.0).astype(jnp.bfloat16)
            acc[b, pl.ds(0, SQ), pl.ds(h * DH, DH)] = jnp.dot(
                w, v_bh, preferred_element_type=jnp.float32)
            acc[b, pl.ds(SQ + h % 2, 1), pl.ds((h // 2) * QW, SQ)] = (
                lax.dot_general(ones_row, w, (((1,), (1,)), ((), ())),
                                preferred_element_type=jnp.float32))

        def exchange(k, q):
            b, hp = q // 2, q % 2
            stage[b, :, pl.ds(hp * QW, QW)] = (
                acc[b, :, pl.ds(hp * QW, QW)].astype(jnp.bfloat16))
            r = pltpu.make_async_remote_copy(
                src_ref=stage.at[b, :, pl.ds(hp * QW, QW)],
                dst_ref=comm.at[k, q],
                send_sem=send_sems.at[k, q], recv_sem=recv_sems.at[k, q],
                device_id=(partners[(k + q) % N_ROUNDS],),
                device_id_type=pl.DeviceIdType.MESH)
            r.start()
            return r

        def accumulate(k, q):
            b, hp = q // 2, q % 2
            acc[b, :, pl.ds(hp * QW, QW)] += comm[k, q].astype(jnp.float32)

        def project(q):
            b, hp = q // 2, q % 2
            s_pair = jnp.transpose(
                acc[b, pl.ds(SQ, 2), pl.ds(hp * QW, SQ)])
            s_rep = jnp.dot(s_pair.astype(jnp.bfloat16), expand,
                            preferred_element_type=jnp.float32)
            ctx = acc[b, pl.ds(0, SQ), pl.ds(hp * QW, QW)] / s_rep
            return jnp.dot(ctx.astype(jnp.bfloat16),
                           wo_ref[pl.ds(hp * QW, QW), :].astype(jnp.bfloat16),
                           preferred_element_type=jnp.float32)

        acc[0, pl.ds(SQ, 8), :] = jnp.zeros((8, COLS), jnp.float32)
        q_b0 = jnp.dot(x_ref[0].astype(jnp.bfloat16),
                       wq_ref[...].astype(jnp.bfloat16),
                       preferred_element_type=jnp.float32)
        compute_head(0, 0, q_b0)
        compute_head(0, 1, q_b0)

        barrier = pltpu.get_barrier_semaphore()
        for p in partners:
            pl.semaphore_signal(barrier, inc=1, device_id=(p,),
                                device_id_type=pl.DeviceIdType.MESH)
        pl.semaphore_wait(barrier, N_ROUNDS)

        rdmas = [None] * NQ
        rdmas[0] = exchange(0, 0)
        compute_head(0, 2, q_b0)
        compute_head(0, 3, q_b0)
        rdmas[1] = exchange(0, 1)

        acc[1, pl.ds(SQ, 8), :] = jnp.zeros((8, COLS), jnp.float32)
        q_b1 = jnp.dot(x_ref[1].astype(jnp.bfloat16),
                       wq_ref[...].astype(jnp.bfloat16),
                       preferred_element_type=jnp.float32)
        compute_head(1, 0, q_b1)
        compute_head(1, 1, q_b1)
        rdmas[2] = exchange(0, 2)
        compute_head(1, 2, q_b1)
        compute_head(1, 3, q_b1)
        rdmas[3] = exchange(0, 3)

        half_out = [None, None]
        for k in range(N_ROUNDS):
            for q in range(NQ):
                rdmas[q].wait()
                accumulate(k, q)
                if k + 1 < N_ROUNDS:
                    rdmas[q] = exchange(k + 1, q)
                elif q % 2 == 0:
                    half_out[q // 2] = project(q)
                else:
                    out_ref[q // 2] = half_out[q // 2] + project(q)

    return pl.pallas_call(
        body,
        out_shape=jax.ShapeDtypeStruct((B, SQ, D_MODEL), jnp.float32),
        in_specs=[pl.BlockSpec(memory_space=pltpu.VMEM)] * 5,
        out_specs=pl.BlockSpec(memory_space=pltpu.VMEM),
        scratch_shapes=[
            pltpu.VMEM((B, PROWS, COLS), jnp.float32),
            pltpu.VMEM((B, PROWS, COLS), jnp.bfloat16),
            pltpu.VMEM((N_ROUNDS, NQ, PROWS, QW), jnp.bfloat16),
            pltpu.SemaphoreType.DMA((N_ROUNDS, NQ)),
            pltpu.SemaphoreType.DMA((N_ROUNDS, NQ)),
        ],
        compiler_params=pltpu.CompilerParams(collective_id=0),
    )(x, Wq, K_ext, V_ext, Wo)
